# Initial kernel scaffold; baseline (speedup 1.0000x reference)
import jax
import jax.numpy as jnp
from jax.experimental import pallas as pl


def kernel(queries, centers):
    raise NotImplementedError("write your pallas kernel here")


# speedup vs baseline: 1.9541x; 1.9541x over previous
import functools

import jax
import jax.numpy as jnp
from jax import lax
from jax.experimental import pallas as pl
from jax.experimental.pallas import tpu as pltpu

_K = 10
_KPAD = 16
_NEG = float("-inf")
_BIG = 2**31 - 1


def _topk_cols(vals, idx, k, kpad):
    q = vals.shape[0]
    col = lax.broadcasted_iota(jnp.int32, (q, kpad), 1)
    out_v = jnp.full((q, kpad), _NEG, jnp.float32)
    out_i = jnp.zeros((q, kpad), jnp.int32)
    for j in range(k):
        m = jnp.max(vals, axis=1, keepdims=True)
        sel = jnp.min(jnp.where(vals == m, idx, _BIG), axis=1,
                      keepdims=True)
        out_v = jnp.where(col == j, m, out_v)
        out_i = jnp.where(col == j, sel, out_i)
        vals = jnp.where(idx == sel, _NEG, vals)
    return out_v, out_i


def _body(c_total, n_tiles, cb, q_ref, c_ref, vals_ref, idx_ref,
          qn_s, rv_s, ri_s):
    t = pl.program_id(0)

    @pl.when(t == 0)
    def _init():
        q = q_ref[...]
        qn_s[...] = q / (jnp.sqrt(jnp.sum(q * q, axis=1, keepdims=True))
                         + 1e-8)
        rv_s[...] = jnp.full(rv_s.shape, _NEG, jnp.float32)
        ri_s[...] = jnp.zeros(ri_s.shape, jnp.int32)

    c = c_ref[...]
    cn = c / (jnp.sqrt(jnp.sum(c * c, axis=1, keepdims=True)) + 1e-8)
    sims = lax.dot_general(qn_s[...], cn, (((1,), (1,)), ((), ())),
                           preferred_element_type=jnp.float32)
    gid = t * cb + lax.broadcasted_iota(jnp.int32, sims.shape, 1)
    sims = jnp.where(gid < c_total, sims, _NEG)

    tv, ti = _topk_cols(sims, gid, _K, _KPAD)
    cv = jnp.concatenate([rv_s[...], tv], axis=1)
    ci = jnp.concatenate([ri_s[...], ti], axis=1)
    nv, ni = _topk_cols(cv, ci, _K, _KPAD)
    rv_s[...] = nv
    ri_s[...] = ni
    vals_ref[...] = nv
    idx_ref[...] = ni


def kernel(queries, centers):
    q_n, d = queries.shape
    c_total = centers.shape[0]
    cb = 2048
    n_tiles = (c_total + cb - 1) // cb
    c_pad = n_tiles * cb
    if c_pad != c_total:
        centers = jnp.pad(centers, ((0, c_pad - c_total), (0, 0)))

    vals, idx = pl.pallas_call(
        functools.partial(_body, c_total, n_tiles, cb),
        grid=(n_tiles,),
        in_specs=[
            pl.BlockSpec((q_n, d), lambda t: (0, 0)),
            pl.BlockSpec((cb, d), lambda t: (t, 0)),
        ],
        out_specs=[
            pl.BlockSpec((q_n, _KPAD), lambda t: (0, 0)),
            pl.BlockSpec((q_n, _KPAD), lambda t: (0, 0)),
        ],
        out_shape=[
            jax.ShapeDtypeStruct((q_n, _KPAD), jnp.float32),
            jax.ShapeDtypeStruct((q_n, _KPAD), jnp.int32),
        ],
        scratch_shapes=[
            pltpu.VMEM((q_n, d), jnp.float32),
            pltpu.VMEM((q_n, _KPAD), jnp.float32),
            pltpu.VMEM((q_n, _KPAD), jnp.int32),
        ],
    )(queries, centers)
    return vals[:, :_K], idx[:, :_K]


# speedup vs baseline: 5.1361x; 2.6283x over previous
import functools

import jax
import jax.numpy as jnp
from jax import lax
from jax.experimental import pallas as pl
from jax.experimental.pallas import tpu as pltpu
from jax.experimental.pallas import tpu_sc as plsc

_K = 10
_KPAD = 16
_G = 128
_NEG = float("-inf")
_BIG = 2**31 - 1


def _sims_body(c_total, cb, gpt, q_ref, c_ref, sims_ref, gmax_ref, qn_s):
    t = pl.program_id(0)

    @pl.when(t == 0)
    def _init():
        q = q_ref[...]
        qn_s[...] = q / (jnp.sqrt(jnp.sum(q * q, axis=1, keepdims=True))
                         + 1e-8)

    c = c_ref[...]
    cn = c / (jnp.sqrt(jnp.sum(c * c, axis=1, keepdims=True)) + 1e-8)
    sims = lax.dot_general(qn_s[...], cn, (((1,), (1,)), ((), ())),
                           preferred_element_type=jnp.float32)
    gid = t * cb + lax.broadcasted_iota(jnp.int32, sims.shape, 1)
    sims = jnp.where(gid < c_total, sims, _NEG)

    qn = sims.shape[0]
    col = lax.broadcasted_iota(jnp.int32, (qn, gpt), 1)
    acc = jnp.full((qn, gpt), _NEG, jnp.float32)
    for g in range(gpt):
        s = sims[:, g * _G:(g + 1) * _G]
        sims_ref[:, g, :] = s
        m = jnp.max(s, axis=1, keepdims=True)
        acc = jnp.where(col == g, m, acc)
    gmax_ref[0] = acc


def _select_body(ng, gmax_ref, sel_ref, flat_ref):
    x = gmax_ref[...]
    qn = x.shape[0]
    giota = lax.broadcasted_iota(jnp.int32, x.shape, 1)
    col = lax.broadcasted_iota(jnp.int32, (qn, _KPAD), 1)
    qrow = lax.broadcasted_iota(jnp.int32, (qn, _KPAD), 0)
    sel = jnp.zeros((qn, _KPAD), jnp.int32)
    for j in range(_K):
        m = jnp.max(x, axis=1, keepdims=True)
        g = jnp.min(jnp.where(x == m, giota, _BIG), axis=1, keepdims=True)
        sel = jnp.where(col == j, g, sel)
        x = jnp.where(giota == g, _NEG, x)
    sel_ref[...] = sel
    flat_ref[...] = qrow * ng + sel


def _merge_body(c_total, cand_ref, sel_ref, vals_ref, idx_ref):
    x = cand_ref[...]
    sel = sel_ref[...]
    qb = x.shape[0]
    lane3 = lax.broadcasted_iota(jnp.int32, x.shape, 2)
    col3 = lax.broadcasted_iota(jnp.int32, x.shape, 1)
    gidx = sel[:, :, None] * _G + lane3
    x = jnp.where((col3 < _K) & (gidx < c_total), x, _NEG)
    col = lax.broadcasted_iota(jnp.int32, (qb, _KPAD), 1)
    outv = jnp.full((qb, _KPAD), _NEG, jnp.float32)
    outi = jnp.zeros((qb, _KPAD), jnp.int32)
    for j in range(_K):
        m = jnp.max(x, axis=(1, 2), keepdims=True)
        si = jnp.min(jnp.where(x == m, gidx, _BIG), axis=(1, 2),
                     keepdims=True)
        outv = jnp.where(col == j, m[:, :, 0], outv)
        outi = jnp.where(col == j, si[:, :, 0], outi)
        x = jnp.where(gidx == si, _NEG, x)
    vals_ref[...] = outv
    idx_ref[...] = outi


def _run_sims(queries, centers, c_total, cb, gpt):
    q_n, d = queries.shape
    n_tiles = centers.shape[0] // cb
    ng = n_tiles * gpt
    return pl.pallas_call(
        functools.partial(_sims_body, c_total, cb, gpt),
        grid=(n_tiles,),
        in_specs=[
            pl.BlockSpec((q_n, d), lambda t: (0, 0)),
            pl.BlockSpec((cb, d), lambda t: (t, 0)),
        ],
        out_specs=[
            pl.BlockSpec((q_n, gpt, _G), lambda t: (0, t, 0)),
            pl.BlockSpec((1, q_n, gpt), lambda t: (t, 0, 0)),
        ],
        out_shape=[
            jax.ShapeDtypeStruct((q_n, ng, _G), jnp.float32),
            jax.ShapeDtypeStruct((n_tiles, q_n, gpt), jnp.float32),
        ],
        scratch_shapes=[pltpu.VMEM((q_n, d), jnp.float32)],
    )(queries, centers)


def _run_select(gmax):
    q_n, ng = gmax.shape
    return pl.pallas_call(
        functools.partial(_select_body, ng),
        out_shape=[
            jax.ShapeDtypeStruct((q_n, _KPAD), jnp.int32),
            jax.ShapeDtypeStruct((q_n, _KPAD), jnp.int32),
        ],
    )(gmax)


def _run_gather(table, flat_idx):
    n_rows = flat_idx.shape[0]
    info = plsc.get_sparse_core_info()
    nc, ns = info.num_cores, info.num_subcores
    nw = nc * ns
    b_per_w = n_rows // nw
    chunks = b_per_w // _G
    idx3 = flat_idx.reshape(nw, chunks, _G)
    mesh = plsc.VectorSubcoreMesh(core_axis_name="c", subcore_axis_name="s")

    @functools.partial(
        pl.kernel, mesh=mesh,
        out_type=jax.ShapeDtypeStruct((n_rows, _G), jnp.float32),
        scratch_types=[
            pltpu.VMEM((chunks, _G), jnp.int32),
            pltpu.VMEM((b_per_w, _G), jnp.float32),
            pltpu.SemaphoreType.DMA,
        ],
    )
    def _gather(table_hbm, idx_hbm, out_hbm, idx_v, rows_v, sem):
        wid = lax.axis_index("s") * nc + lax.axis_index("c")
        pltpu.sync_copy(idx_hbm.at[wid], idx_v)
        cps = [
            pltpu.async_copy(table_hbm.at[idx_v.at[j]],
                             rows_v.at[pl.ds(j * _G, _G)], sem)
            for j in range(chunks)
        ]
        for cp in cps:
            cp.wait()
        pltpu.sync_copy(rows_v, out_hbm.at[pl.ds(wid * b_per_w, b_per_w)])

    return _gather(table, idx3)


def _run_merge(cand3, sel, c_total):
    q_n = sel.shape[0]
    qb = min(128, q_n)
    return pl.pallas_call(
        functools.partial(_merge_body, c_total),
        grid=(q_n // qb,),
        in_specs=[
            pl.BlockSpec((qb, _KPAD, _G), lambda i: (i, 0, 0)),
            pl.BlockSpec((qb, _KPAD), lambda i: (i, 0)),
        ],
        out_specs=[
            pl.BlockSpec((qb, _KPAD), lambda i: (i, 0)),
            pl.BlockSpec((qb, _KPAD), lambda i: (i, 0)),
        ],
        out_shape=[
            jax.ShapeDtypeStruct((q_n, _KPAD), jnp.float32),
            jax.ShapeDtypeStruct((q_n, _KPAD), jnp.int32),
        ],
    )(cand3, sel)


def kernel(queries, centers):
    q_n, d = queries.shape
    c_total = centers.shape[0]
    cb = 2048
    gpt = cb // _G
    n_tiles = (c_total + cb - 1) // cb
    c_pad = n_tiles * cb
    ng = n_tiles * gpt
    if c_pad != c_total:
        centers = jnp.pad(centers, ((0, c_pad - c_total), (0, 0)))

    sims3, gmax = _run_sims(queries, centers, c_total, cb, gpt)
    sel, flat = _run_select(gmax.transpose(1, 0, 2).reshape(q_n, ng))
    cand = _run_gather(sims3.reshape(q_n * ng, _G), flat.reshape(-1))
    vals, idx = _run_merge(cand.reshape(q_n, _KPAD, _G), sel, c_total)
    return vals[:, :_K], idx[:, :_K]


# speedup vs baseline: 8.2062x; 1.5978x over previous
import functools

import jax
import jax.numpy as jnp
from jax import lax
from jax.experimental import pallas as pl
from jax.experimental.pallas import tpu as pltpu
from jax.experimental.pallas import tpu_sc as plsc

_K = 10
_KPAD = 16
_G = 128
_NEG = float("-inf")
_BIG = 2**31 - 1


def _sims_body(c_total, cb, gpt, q_ref, c_ref, sims_ref, gmax_ref, qn_s):
    t = pl.program_id(0)

    @pl.when(t == 0)
    def _init():
        q = q_ref[...]
        qn_s[...] = q / (jnp.sqrt(jnp.sum(q * q, axis=1, keepdims=True))
                         + 1e-8)

    c = c_ref[...]
    cn = c / (jnp.sqrt(jnp.sum(c * c, axis=1, keepdims=True)) + 1e-8)
    sims = lax.dot_general(qn_s[...], cn, (((1,), (1,)), ((), ())),
                           preferred_element_type=jnp.float32)
    gid = t * cb + lax.broadcasted_iota(jnp.int32, sims.shape, 1)
    sims = jnp.where(gid < c_total, sims, _NEG)

    qn = sims.shape[0]
    col = lax.broadcasted_iota(jnp.int32, (qn, gpt), 1)
    acc = jnp.full((qn, gpt), _NEG, jnp.float32)
    for g in range(gpt):
        s = sims[:, g * _G:(g + 1) * _G]
        sims_ref[g] = s
        m = jnp.max(s, axis=1, keepdims=True)
        acc = jnp.where(col == g, m, acc)
    gmax_ref[0] = acc


def _select_body(ng, gmax_ref, sel_ref, flat_ref):
    x = gmax_ref[...]
    qn = x.shape[0]
    giota = lax.broadcasted_iota(jnp.int32, x.shape, 1)
    col = lax.broadcasted_iota(jnp.int32, (qn, _KPAD), 1)
    qrow = lax.broadcasted_iota(jnp.int32, (qn, _KPAD), 0)
    sel = jnp.zeros((qn, _KPAD), jnp.int32)
    for j in range(_K):
        m = jnp.max(x, axis=1, keepdims=True)
        g = jnp.min(jnp.where(x == m, giota, _BIG), axis=1, keepdims=True)
        sel = jnp.where(col == j, g, sel)
        x = jnp.where(giota == g, _NEG, x)
    sel_ref[...] = sel
    flat_ref[...] = sel * qn + qrow


def _merge_body(c_total, cand_ref, sel_ref, vals_ref, idx_ref):
    x = cand_ref[...]
    sel = sel_ref[...]
    qb = x.shape[0]
    lane3 = lax.broadcasted_iota(jnp.int32, x.shape, 2)
    col3 = lax.broadcasted_iota(jnp.int32, x.shape, 1)
    gidx = sel[:, :, None] * _G + lane3
    x = jnp.where((col3 < _K) & (gidx < c_total), x, _NEG)
    col = lax.broadcasted_iota(jnp.int32, (qb, _KPAD), 1)
    outv = jnp.full((qb, _KPAD), _NEG, jnp.float32)
    outi = jnp.zeros((qb, _KPAD), jnp.int32)
    for j in range(_K):
        m = jnp.max(x, axis=(1, 2), keepdims=True)
        si = jnp.min(jnp.where(x == m, gidx, _BIG), axis=(1, 2),
                     keepdims=True)
        outv = jnp.where(col == j, m[:, :, 0], outv)
        outi = jnp.where(col == j, si[:, :, 0], outi)
        x = jnp.where(gidx == si, _NEG, x)
    vals_ref[...] = outv
    idx_ref[...] = outi


def _run_sims(queries, centers, c_total, cb, gpt):
    q_n, d = queries.shape
    n_tiles = centers.shape[0] // cb
    ng = n_tiles * gpt
    return pl.pallas_call(
        functools.partial(_sims_body, c_total, cb, gpt),
        grid=(n_tiles,),
        in_specs=[
            pl.BlockSpec((q_n, d), lambda t: (0, 0)),
            pl.BlockSpec((cb, d), lambda t: (t, 0)),
        ],
        out_specs=[
            pl.BlockSpec((gpt, q_n, _G), lambda t: (t, 0, 0)),
            pl.BlockSpec((1, q_n, gpt), lambda t: (t, 0, 0)),
        ],
        out_shape=[
            jax.ShapeDtypeStruct((ng, q_n, _G), jnp.float32),
            jax.ShapeDtypeStruct((n_tiles, q_n, gpt), jnp.float32),
        ],
        scratch_shapes=[pltpu.VMEM((q_n, d), jnp.float32)],
    )(queries, centers)


def _run_select(gmax):
    q_n, ng = gmax.shape
    return pl.pallas_call(
        functools.partial(_select_body, ng),
        out_shape=[
            jax.ShapeDtypeStruct((q_n, _KPAD), jnp.int32),
            jax.ShapeDtypeStruct((q_n, _KPAD), jnp.int32),
        ],
    )(gmax)


def _run_gather(table, flat_idx):
    n_rows = flat_idx.shape[0]
    info = plsc.get_sparse_core_info()
    nc, ns = info.num_cores, info.num_subcores
    nw = nc * ns
    b_per_w = n_rows // nw
    chunks = b_per_w // _G
    idx3 = flat_idx.reshape(nw, chunks, _G)
    mesh = plsc.VectorSubcoreMesh(core_axis_name="c", subcore_axis_name="s")

    @functools.partial(
        pl.kernel, mesh=mesh,
        out_type=jax.ShapeDtypeStruct((n_rows, _G), jnp.float32),
        scratch_types=[
            pltpu.VMEM((chunks, _G), jnp.int32),
            pltpu.VMEM((b_per_w, _G), jnp.float32),
            pltpu.SemaphoreType.DMA,
        ],
    )
    def _gather(table_hbm, idx_hbm, out_hbm, idx_v, rows_v, sem):
        wid = lax.axis_index("s") * nc + lax.axis_index("c")
        pltpu.sync_copy(idx_hbm.at[wid], idx_v)
        cps = [
            pltpu.async_copy(table_hbm.at[idx_v.at[j]],
                             rows_v.at[pl.ds(j * _G, _G)], sem)
            for j in range(chunks)
        ]
        for cp in cps:
            cp.wait()
        pltpu.sync_copy(rows_v, out_hbm.at[pl.ds(wid * b_per_w, b_per_w)])

    return _gather(table, idx3)


def _run_merge(cand3, sel, c_total):
    q_n = sel.shape[0]
    qb = min(128, q_n)
    return pl.pallas_call(
        functools.partial(_merge_body, c_total),
        grid=(q_n // qb,),
        in_specs=[
            pl.BlockSpec((qb, _KPAD, _G), lambda i: (i, 0, 0)),
            pl.BlockSpec((qb, _KPAD), lambda i: (i, 0)),
        ],
        out_specs=[
            pl.BlockSpec((qb, _KPAD), lambda i: (i, 0)),
            pl.BlockSpec((qb, _KPAD), lambda i: (i, 0)),
        ],
        out_shape=[
            jax.ShapeDtypeStruct((q_n, _KPAD), jnp.float32),
            jax.ShapeDtypeStruct((q_n, _KPAD), jnp.int32),
        ],
    )(cand3, sel)


def kernel(queries, centers):
    q_n, d = queries.shape
    c_total = centers.shape[0]
    cb = 2048
    gpt = cb // _G
    n_tiles = (c_total + cb - 1) // cb
    c_pad = n_tiles * cb
    ng = n_tiles * gpt
    if c_pad != c_total:
        centers = jnp.pad(centers, ((0, c_pad - c_total), (0, 0)))

    sims3, gmax = _run_sims(queries, centers, c_total, cb, gpt)
    sel, flat = _run_select(gmax.transpose(1, 0, 2).reshape(q_n, ng))
    cand = _run_gather(sims3.reshape(ng * q_n, _G), flat.reshape(-1))
    vals, idx = _run_merge(cand.reshape(q_n, _KPAD, _G), sel, c_total)
    return vals[:, :_K], idx[:, :_K]


# speedup vs baseline: 8.5083x; 1.0368x over previous
import functools

import jax
import jax.numpy as jnp
from jax import lax
from jax.experimental import pallas as pl
from jax.experimental.pallas import tpu as pltpu
from jax.experimental.pallas import tpu_sc as plsc

_K = 10
_KPAD = 16
_G = 128
_NEG = float("-inf")
_BIG = 2**31 - 1


def _sims_body(c_total, cb, gpt, q_ref, c_ref, sims_ref, gmax_ref, qn_s):
    t = pl.program_id(0)

    @pl.when(t == 0)
    def _init():
        q = q_ref[...]
        qn_s[...] = q / (jnp.sqrt(jnp.sum(q * q, axis=1, keepdims=True))
                         + 1e-8)

    c = c_ref[...]
    cn = c / (jnp.sqrt(jnp.sum(c * c, axis=1, keepdims=True)) + 1e-8)
    sims = lax.dot_general(qn_s[...], cn, (((1,), (1,)), ((), ())),
                           preferred_element_type=jnp.float32)
    gid = t * cb + lax.broadcasted_iota(jnp.int32, sims.shape, 1)
    sims = jnp.where(gid < c_total, sims, _NEG)

    qn = sims.shape[0]
    col = lax.broadcasted_iota(jnp.int32, (qn, gpt), 1)
    acc = jnp.full((qn, gpt), _NEG, jnp.float32)
    for g in range(gpt):
        s = sims[:, g * _G:(g + 1) * _G]
        sims_ref[g] = s
        m = jnp.max(s, axis=1, keepdims=True)
        acc = jnp.where(col == g, m, acc)
    gmax_ref[0] = acc


def _select_body(ng, gmax_ref, sel_ref, flat_ref):
    x = gmax_ref[...]
    qn = x.shape[0]
    giota = lax.broadcasted_iota(jnp.int32, x.shape, 1)
    col = lax.broadcasted_iota(jnp.int32, (qn, _KPAD), 1)
    qrow = lax.broadcasted_iota(jnp.int32, (qn, _KPAD), 0)
    sel = jnp.zeros((qn, _KPAD), jnp.int32)
    for j in range(_K):
        m = jnp.max(x, axis=1, keepdims=True)
        g = jnp.min(jnp.where(x == m, giota, _BIG), axis=1, keepdims=True)
        sel = jnp.where(col == j, g, sel)
        x = jnp.where(giota == g, _NEG, x)
    sel_ref[...] = sel
    flat_ref[...] = sel * qn + qrow


def _merge_body(c_total, cand_ref, sel_ref, vals_ref, idx_ref):
    x = cand_ref[...]
    sel = sel_ref[...]
    qb = x.shape[0]
    lane3 = lax.broadcasted_iota(jnp.int32, x.shape, 2)
    col3 = lax.broadcasted_iota(jnp.int32, x.shape, 1)
    gidx = sel[:, :, None] * _G + lane3
    x = jnp.where((col3 < _K) & (gidx < c_total), x, _NEG)
    col = lax.broadcasted_iota(jnp.int32, (qb, _KPAD), 1)
    outv = jnp.full((qb, _KPAD), _NEG, jnp.float32)
    outi = jnp.zeros((qb, _KPAD), jnp.int32)
    for j in range(_K):
        m = jnp.max(x, axis=(1, 2), keepdims=True)
        si = jnp.min(jnp.where(x == m, gidx, _BIG), axis=(1, 2),
                     keepdims=True)
        outv = jnp.where(col == j, m[:, :, 0], outv)
        outi = jnp.where(col == j, si[:, :, 0], outi)
        x = jnp.where(gidx == si, _NEG, x)
    vals_ref[...] = outv
    idx_ref[...] = outi


def _run_sims(queries, centers, c_total, cb, gpt):
    q_n, d = queries.shape
    n_tiles = centers.shape[0] // cb
    ng = n_tiles * gpt
    return pl.pallas_call(
        functools.partial(_sims_body, c_total, cb, gpt),
        grid=(n_tiles,),
        in_specs=[
            pl.BlockSpec((q_n, d), lambda t: (0, 0)),
            pl.BlockSpec((cb, d), lambda t: (t, 0)),
        ],
        out_specs=[
            pl.BlockSpec((gpt, q_n, _G), lambda t: (t, 0, 0)),
            pl.BlockSpec((1, q_n, gpt), lambda t: (t, 0, 0)),
        ],
        out_shape=[
            jax.ShapeDtypeStruct((ng, q_n, _G), jnp.float32),
            jax.ShapeDtypeStruct((n_tiles, q_n, gpt), jnp.float32),
        ],
        scratch_shapes=[pltpu.VMEM((q_n, d), jnp.float32)],
    )(queries, centers)


def _run_select(gmax):
    q_n, ng = gmax.shape
    return pl.pallas_call(
        functools.partial(_select_body, ng),
        out_shape=[
            jax.ShapeDtypeStruct((q_n, _KPAD), jnp.int32),
            jax.ShapeDtypeStruct((q_n, _KPAD), jnp.int32),
        ],
    )(gmax)


def _run_gather(table, flat_idx):
    n_rows = flat_idx.shape[0]
    info = plsc.get_sparse_core_info()
    nc, ns = info.num_cores, info.num_subcores
    nw = nc * ns
    b_per_w = n_rows // nw
    chunks = b_per_w // _G
    idx3 = flat_idx.reshape(nw, chunks, _G)
    mesh = plsc.VectorSubcoreMesh(core_axis_name="c", subcore_axis_name="s")

    @functools.partial(
        pl.kernel, mesh=mesh,
        out_type=jax.ShapeDtypeStruct((n_rows, _G), jnp.float32),
        scratch_types=[
            pltpu.VMEM((chunks, _G), jnp.int32),
            pltpu.VMEM((b_per_w, _G), jnp.float32),
            pltpu.SemaphoreType.DMA,
        ],
    )
    def _gather(table_hbm, idx_hbm, out_hbm, idx_v, rows_v, sem):
        wid = lax.axis_index("s") * nc + lax.axis_index("c")
        pltpu.sync_copy(idx_hbm.at[wid], idx_v)
        cps = [
            pltpu.async_copy(table_hbm.at[idx_v.at[j]],
                             rows_v.at[pl.ds(j * _G, _G)], sem)
            for j in range(chunks)
        ]
        for cp in cps:
            cp.wait()
        pltpu.sync_copy(rows_v, out_hbm.at[pl.ds(wid * b_per_w, b_per_w)])

    return _gather(table, idx3)


def _run_merge(cand3, sel, c_total):
    q_n = sel.shape[0]
    qb = min(128, q_n)
    return pl.pallas_call(
        functools.partial(_merge_body, c_total),
        grid=(q_n // qb,),
        in_specs=[
            pl.BlockSpec((qb, _KPAD, _G), lambda i: (i, 0, 0)),
            pl.BlockSpec((qb, _KPAD), lambda i: (i, 0)),
        ],
        out_specs=[
            pl.BlockSpec((qb, _KPAD), lambda i: (i, 0)),
            pl.BlockSpec((qb, _KPAD), lambda i: (i, 0)),
        ],
        out_shape=[
            jax.ShapeDtypeStruct((q_n, _KPAD), jnp.float32),
            jax.ShapeDtypeStruct((q_n, _KPAD), jnp.int32),
        ],
    )(cand3, sel)


def kernel(queries, centers):
    q_n, d = queries.shape
    c_total = centers.shape[0]
    cb = 4096
    gpt = cb // _G
    n_tiles = (c_total + cb - 1) // cb
    c_pad = n_tiles * cb
    ng = n_tiles * gpt
    if c_pad != c_total:
        centers = jnp.pad(centers, ((0, c_pad - c_total), (0, 0)))

    sims3, gmax = _run_sims(queries, centers, c_total, cb, gpt)
    sel, flat = _run_select(gmax.transpose(1, 0, 2).reshape(q_n, ng))
    cand = _run_gather(sims3.reshape(ng * q_n, _G), flat.reshape(-1))
    vals, idx = _run_merge(cand.reshape(q_n, _KPAD, _G), sel, c_total)
    return vals[:, :_K], idx[:, :_K]
